# Initial kernel scaffold; baseline (speedup 1.0000x reference)
import jax
import jax.numpy as jnp
from jax.experimental import pallas as pl


def kernel(x, embeddings, W0, W1, W2):
    raise NotImplementedError("write your pallas kernel here")


# speedup vs baseline: 177.9324x; 177.9324x over previous
import numpy as np
import jax
import jax.numpy as jnp
from jax import lax
from jax.experimental import pallas as pl
from jax.experimental.pallas import tpu as pltpu
from jax.experimental.pallas import tpu_sc as plsc

_NUM_LEVELS = 16
_T = 2 ** 14
_BASE_RES = 16
_SCALE = float(np.exp2(np.log2(1024 / 16) / (_NUM_LEVELS - 1)))
_N = 524288
_P2 = int(np.array([2654435761], np.uint32).view(np.int32)[0])
_P3 = 805459861

_RES = [int(np.floor(_BASE_RES * _SCALE ** l)) for l in range(_NUM_LEVELS)]
_DENSE = [(r + 1) ** 3 <= _T for r in _RES]
_MY = [((r + 1) if d else _P2) for r, d in zip(_RES, _DENSE)]
_MZ = [((r + 1) ** 2 if d else _P3) for r, d in zip(_RES, _DENSE)]

_NC, _NS = 2, 16
_B = 4096


def _sc_body(x0, x1, x2, embf, pmy, pmz, pdense, presf, out,
             tab0, tab1, tab2, tab3, xb0, xb1, xb2, fb0, fb1, fb2, fb3,
             pmyv, pmzv, pdensev, presfv):
    c = lax.axis_index("c")
    level = lax.axis_index("s")
    ebase = level * (4 * _T)
    pltpu.sync_copy(embf.at[pl.ds(ebase + 0 * _T, _T)], tab0)
    pltpu.sync_copy(embf.at[pl.ds(ebase + 1 * _T, _T)], tab1)
    pltpu.sync_copy(embf.at[pl.ds(ebase + 2 * _T, _T)], tab2)
    pltpu.sync_copy(embf.at[pl.ds(ebase + 3 * _T, _T)], tab3)
    pltpu.sync_copy(pmy, pmyv)
    pltpu.sync_copy(pmz, pmzv)
    pltpu.sync_copy(pdense, pdensev)
    pltpu.sync_copy(presf, presfv)
    lvl = jnp.full((16,), level, dtype=jnp.int32)
    my = plsc.load_gather(pmyv, [lvl])
    mz = plsc.load_gather(pmzv, [lvl])
    dense = plsc.load_gather(pdensev, [lvl]) > 0
    resf = plsc.load_gather(presfv, [lvl])
    half_n = _N // _NC
    base0 = c * half_n
    mask = jnp.int32(_T - 1)

    def block(b, carry):
        base = base0 + b * _B
        pltpu.sync_copy(x0.at[pl.ds(base, _B)], xb0)
        pltpu.sync_copy(x1.at[pl.ds(base, _B)], xb1)
        pltpu.sync_copy(x2.at[pl.ds(base, _B)], xb2)

        def vstep(i, carry2):
            off = i * 16
            px = xb0[pl.ds(off, 16)] * resf
            py = xb1[pl.ds(off, 16)] * resf
            pz = xb2[pl.ds(off, 16)] * resf
            ix = px.astype(jnp.int32)
            iy = py.astype(jnp.int32)
            iz = pz.astype(jnp.int32)
            fx = px - ix.astype(jnp.float32)
            fy = py - iy.astype(jnp.float32)
            fz = pz - iz.astype(jnp.float32)
            wx = (1.0 - fx, fx)
            wy = (1.0 - fy, fy)
            wz = (1.0 - fz, fz)
            ax = (ix, ix + 1)
            ay0 = iy * my
            az0 = iz * mz
            ay = (ay0, ay0 + my)
            az = (az0, az0 + mz)
            ayz = [[None, None], [None, None]]
            for dy in (0, 1):
                for dz in (0, 1):
                    ayz[dy][dz] = jnp.where(dense, ay[dy] + az[dz],
                                            ay[dy] ^ az[dz])
            acc0 = acc1 = acc2 = acc3 = None
            for dx in (0, 1):
                for dy in (0, 1):
                    wxy = wx[dx] * wy[dy]
                    for dz in (0, 1):
                        idx = jnp.where(dense, ax[dx] + ayz[dy][dz],
                                        ax[dx] ^ ayz[dy][dz]) & mask
                        w = wxy * wz[dz]
                        g0 = plsc.load_gather(tab0, [idx])
                        g1 = plsc.load_gather(tab1, [idx])
                        g2 = plsc.load_gather(tab2, [idx])
                        g3 = plsc.load_gather(tab3, [idx])
                        if acc0 is None:
                            acc0 = w * g0
                            acc1 = w * g1
                            acc2 = w * g2
                            acc3 = w * g3
                        else:
                            acc0 = acc0 + w * g0
                            acc1 = acc1 + w * g1
                            acc2 = acc2 + w * g2
                            acc3 = acc3 + w * g3
            fb0[pl.ds(off, 16)] = acc0
            fb1[pl.ds(off, 16)] = acc1
            fb2[pl.ds(off, 16)] = acc2
            fb3[pl.ds(off, 16)] = acc3
            return carry2

        lax.fori_loop(0, _B // 16, vstep, 0)
        obase = (level * 4) * _N + base
        pltpu.sync_copy(fb0, out.at[pl.ds(obase + 0 * _N, _B)])
        pltpu.sync_copy(fb1, out.at[pl.ds(obase + 1 * _N, _B)])
        pltpu.sync_copy(fb2, out.at[pl.ds(obase + 2 * _N, _B)])
        pltpu.sync_copy(fb3, out.at[pl.ds(obase + 3 * _N, _B)])
        return carry

    lax.fori_loop(0, half_n // _B, block, 0)


def _encode(x0, x1, x2, embf, pmy, pmz, pdense, presf):
    mesh = plsc.VectorSubcoreMesh(core_axis_name="c", subcore_axis_name="s",
                                  num_cores=_NC, num_subcores=_NS)
    f = pl.kernel(
        _sc_body,
        out_type=jax.ShapeDtypeStruct((4 * _NUM_LEVELS * _N,), jnp.float32),
        mesh=mesh,
        compiler_params=pltpu.CompilerParams(needs_layout_passes=False),
        scratch_types=[
            pltpu.VMEM((_T,), jnp.float32),
            pltpu.VMEM((_T,), jnp.float32),
            pltpu.VMEM((_T,), jnp.float32),
            pltpu.VMEM((_T,), jnp.float32),
            pltpu.VMEM((_B,), jnp.float32),
            pltpu.VMEM((_B,), jnp.float32),
            pltpu.VMEM((_B,), jnp.float32),
            pltpu.VMEM((_B,), jnp.float32),
            pltpu.VMEM((_B,), jnp.float32),
            pltpu.VMEM((_B,), jnp.float32),
            pltpu.VMEM((_B,), jnp.float32),
            pltpu.VMEM((_NS,), jnp.int32),
            pltpu.VMEM((_NS,), jnp.int32),
            pltpu.VMEM((_NS,), jnp.int32),
            pltpu.VMEM((_NS,), jnp.float32),
        ],
    )
    return f(x0, x1, x2, embf, pmy, pmz, pdense, presf)


_BN = 2048


def _mlp_body(xT_ref, hT_ref, a0x_ref, a0h_ref, a1_ref, a2_ref, out_ref):
    xb = xT_ref[...] * 2.0 - 1.0
    dn = (((1,), (0,)), ((), ()))
    z0 = lax.dot_general(a0x_ref[...], xb, dn,
                         preferred_element_type=jnp.float32)
    z0 = z0 + lax.dot_general(a0h_ref[...], hT_ref[...], dn,
                              preferred_element_type=jnp.float32)
    z0 = jnp.maximum(z0, 0.0)
    z1 = jnp.maximum(
        lax.dot_general(a1_ref[...], z0, dn,
                        preferred_element_type=jnp.float32), 0.0)
    out_ref[...] = jnp.abs(
        lax.dot_general(a2_ref[...], z1, dn,
                        preferred_element_type=jnp.float32))


def _mlp(xT, hT, a0x, a0h, a1, a2):
    grid = (_N // _BN,)
    return pl.pallas_call(
        _mlp_body,
        grid=grid,
        in_specs=[
            pl.BlockSpec((3, _BN), lambda i: (0, i)),
            pl.BlockSpec((4 * _NUM_LEVELS, _BN), lambda i: (0, i)),
            pl.BlockSpec((64, 3), lambda i: (0, 0)),
            pl.BlockSpec((64, 64), lambda i: (0, 0)),
            pl.BlockSpec((64, 64), lambda i: (0, 0)),
            pl.BlockSpec((3, 64), lambda i: (0, 0)),
        ],
        out_specs=pl.BlockSpec((3, _BN), lambda i: (0, i)),
        out_shape=jax.ShapeDtypeStruct((3, _N), jnp.float32),
    )(xT, hT, a0x, a0h, a1, a2)


def kernel(x, embeddings, W0, W1, W2):
    xT = x.T
    embf = jnp.transpose(embeddings, (0, 2, 1)).reshape(-1)
    pmy = jnp.asarray(_MY, dtype=jnp.int32)
    pmz = jnp.asarray(_MZ, dtype=jnp.int32)
    pdense = jnp.asarray([1 if d else 0 for d in _DENSE], dtype=jnp.int32)
    presf = jnp.asarray([float(r) for r in _RES], dtype=jnp.float32)
    hT = _encode(xT[0], xT[1], xT[2], embf, pmy, pmz, pdense,
                 presf).reshape(4 * _NUM_LEVELS, _N)
    a0x = W0[:3].T
    a0h = W0[3:].T
    a1 = W1.T
    a2 = W2.T
    outT = _mlp(xT, hT, a0x, a0h, a1, a2)
    return outT.T


# speedup vs baseline: 203.6560x; 1.1446x over previous
import numpy as np
import jax
import jax.numpy as jnp
from jax import lax
from jax.experimental import pallas as pl
from jax.experimental.pallas import tpu as pltpu
from jax.experimental.pallas import tpu_sc as plsc

_NUM_LEVELS = 16
_T = 2 ** 14
_BASE_RES = 16
_SCALE = float(np.exp2(np.log2(1024 / 16) / (_NUM_LEVELS - 1)))
_N = 524288
_P2 = int(np.array([2654435761], np.uint32).view(np.int32)[0])
_P3 = 805459861

_RES = [int(np.floor(_BASE_RES * _SCALE ** l)) for l in range(_NUM_LEVELS)]
_NDENSE = sum(1 for r in _RES if (r + 1) ** 3 <= _T)

_NC, _NS = 2, 16
_B = 4096
_NBLK = _N // _B


def _sc_body(xb3, embf, out, tab0, tab1, tab2, tab3,
             xbuf0, xbuf1, fbuf0, fbuf1, sin0, sin1, sout0, sout1):
    c = lax.axis_index("c")
    level = lax.axis_index("s")
    ebase = level * (4 * _T)
    pltpu.sync_copy(embf.at[pl.ds(ebase + 0 * _T, _T)], tab0)
    pltpu.sync_copy(embf.at[pl.ds(ebase + 1 * _T, _T)], tab1)
    pltpu.sync_copy(embf.at[pl.ds(ebase + 2 * _T, _T)], tab2)
    pltpu.sync_copy(embf.at[pl.ds(ebase + 3 * _T, _T)], tab3)

    nblk = _NBLK // _NC
    gbase = c * nblk
    xbufs = (xbuf0, xbuf1)
    fbufs = (fbuf0, fbuf1)
    sins = (sin0, sin1)
    souts = (sout0, sout1)

    def start_in(g, sl):
        pltpu.make_async_copy(xb3.at[pl.ds(g * (3 * _B), 3 * _B)],
                              xbufs[sl], sins[sl]).start()

    def wait_in(sl):
        pltpu.make_async_copy(xb3.at[pl.ds(0, 3 * _B)],
                              xbufs[sl], sins[sl]).wait()

    def start_out(g, sl):
        dst = out.at[pl.ds(g * (4 * _NUM_LEVELS * _B) + (level * 4) * _B,
                           4 * _B)]
        pltpu.make_async_copy(fbufs[sl], dst, souts[sl]).start()

    def wait_out(sl):
        dst = out.at[pl.ds((level * 4) * _B, 4 * _B)]
        pltpu.make_async_copy(fbufs[sl], dst, souts[sl]).wait()

    def compute(xb, fb, my, mz, resf, dense_path):
        def vstep(i, carry):
            off = i * 16
            px = xb[pl.ds(off, 16)] * resf
            py = xb[pl.ds(_B + off, 16)] * resf
            pz = xb[pl.ds(2 * _B + off, 16)] * resf
            ix = px.astype(jnp.int32)
            iy = py.astype(jnp.int32)
            iz = pz.astype(jnp.int32)
            fx = px - ix.astype(jnp.float32)
            fy = py - iy.astype(jnp.float32)
            fz = pz - iz.astype(jnp.float32)
            wx = (1.0 - fx, fx)
            wy = (1.0 - fy, fy)
            wz = (1.0 - fz, fz)
            ax = (ix, ix + 1)
            ay0 = iy * my
            az0 = iz * mz
            ay = (ay0, ay0 + my)
            az = (az0, az0 + mz)
            ayz = [[None, None], [None, None]]
            for dy in (0, 1):
                for dz in (0, 1):
                    if dense_path:
                        ayz[dy][dz] = ay[dy] + az[dz]
                    else:
                        ayz[dy][dz] = ay[dy] ^ az[dz]
            acc = [None] * 4
            tabs = (tab0, tab1, tab2, tab3)
            for dx in (0, 1):
                for dy in (0, 1):
                    wxy = wx[dx] * wy[dy]
                    for dz in (0, 1):
                        if dense_path:
                            idx = ax[dx] + ayz[dy][dz]
                        else:
                            idx = (ax[dx] ^ ayz[dy][dz]) & jnp.int32(_T - 1)
                        w = wxy * wz[dz]
                        for k in range(4):
                            g = plsc.load_gather(tabs[k], [idx])
                            if acc[k] is None:
                                acc[k] = w * g
                            else:
                                acc[k] = acc[k] + w * g
            fb[pl.ds(0 * _B + off, 16)] = acc[0]
            fb[pl.ds(1 * _B + off, 16)] = acc[1]
            fb[pl.ds(2 * _B + off, 16)] = acc[2]
            fb[pl.ds(3 * _B + off, 16)] = acc[3]
            return carry

        lax.fori_loop(0, _B // 16, vstep, 0, unroll=4)

    def pipeline(dense_path):
        if dense_path:
            resf_s = jnp.where(level == 0, jnp.float32(_RES[0]),
                               jnp.float32(_RES[1]))
            my_s = jnp.where(level == 0, jnp.int32(_RES[0] + 1),
                             jnp.int32(_RES[1] + 1))
            mz_s = jnp.where(level == 0, jnp.int32((_RES[0] + 1) ** 2),
                             jnp.int32((_RES[1] + 1) ** 2))
            my = jnp.full((16,), my_s, dtype=jnp.int32)
            mz = jnp.full((16,), mz_s, dtype=jnp.int32)
        else:
            resf_s = jnp.float32(_RES[_NUM_LEVELS - 1])
            for l in reversed(range(_NDENSE, _NUM_LEVELS - 1)):
                resf_s = jnp.where(level == l, jnp.float32(_RES[l]), resf_s)
            my = jnp.full((16,), _P2, dtype=jnp.int32)
            mz = jnp.full((16,), _P3, dtype=jnp.int32)
        resf = jnp.full((16,), resf_s, dtype=jnp.float32)

        start_in(gbase, 0)

        def outer(b2, carry):
            for sl in (0, 1):
                b = b2 * 2 + sl
                g = gbase + b
                wait_in(sl)

                @pl.when(b + 1 < nblk)
                def _():
                    start_in(g + 1, 1 - sl)

                @pl.when(b >= 2)
                def _():
                    wait_out(sl)

                compute(xbufs[sl], fbufs[sl], my, mz, resf, dense_path)
                start_out(g, sl)
            return carry

        lax.fori_loop(0, nblk // 2, outer, 0)
        wait_out(0)
        wait_out(1)

    lax.cond(level < _NDENSE,
             lambda: pipeline(True),
             lambda: pipeline(False))


def _encode(xb3f, embf):
    mesh = plsc.VectorSubcoreMesh(core_axis_name="c", subcore_axis_name="s",
                                  num_cores=_NC, num_subcores=_NS)
    f = pl.kernel(
        _sc_body,
        out_type=jax.ShapeDtypeStruct((_N * 4 * _NUM_LEVELS,), jnp.float32),
        mesh=mesh,
        compiler_params=pltpu.CompilerParams(needs_layout_passes=False),
        scratch_types=[
            pltpu.VMEM((_T,), jnp.float32),
            pltpu.VMEM((_T,), jnp.float32),
            pltpu.VMEM((_T,), jnp.float32),
            pltpu.VMEM((_T,), jnp.float32),
            pltpu.VMEM((3 * _B,), jnp.float32),
            pltpu.VMEM((3 * _B,), jnp.float32),
            pltpu.VMEM((4 * _B,), jnp.float32),
            pltpu.VMEM((4 * _B,), jnp.float32),
            pltpu.SemaphoreType.DMA,
            pltpu.SemaphoreType.DMA,
            pltpu.SemaphoreType.DMA,
            pltpu.SemaphoreType.DMA,
        ],
    )
    return f(xb3f, embf)


def _mlp_body(xb_ref, hb_ref, a0x_ref, a0h_ref, a1_ref, a2_ref, out_ref):
    xb = xb_ref[0] * 2.0 - 1.0
    dn = (((1,), (0,)), ((), ()))
    z0 = lax.dot_general(a0x_ref[...], xb, dn,
                         preferred_element_type=jnp.float32)
    z0 = z0 + lax.dot_general(a0h_ref[...], hb_ref[0], dn,
                              preferred_element_type=jnp.float32)
    z0 = jnp.maximum(z0, 0.0)
    z1 = jnp.maximum(
        lax.dot_general(a1_ref[...], z0, dn,
                        preferred_element_type=jnp.float32), 0.0)
    out_ref[...] = jnp.abs(
        lax.dot_general(a2_ref[...], z1, dn,
                        preferred_element_type=jnp.float32))


def _mlp(xb3, h3, a0x, a0h, a1, a2):
    grid = (_NBLK,)
    return pl.pallas_call(
        _mlp_body,
        grid=grid,
        in_specs=[
            pl.BlockSpec((1, 3, _B), lambda i: (i, 0, 0)),
            pl.BlockSpec((1, 4 * _NUM_LEVELS, _B), lambda i: (i, 0, 0)),
            pl.BlockSpec((64, 3), lambda i: (0, 0)),
            pl.BlockSpec((64, 64), lambda i: (0, 0)),
            pl.BlockSpec((64, 64), lambda i: (0, 0)),
            pl.BlockSpec((3, 64), lambda i: (0, 0)),
        ],
        out_specs=pl.BlockSpec((3, _B), lambda i: (0, i)),
        out_shape=jax.ShapeDtypeStruct((3, _N), jnp.float32),
    )(xb3, h3, a0x, a0h, a1, a2)


def kernel(x, embeddings, W0, W1, W2):
    xb3 = x.reshape(_NBLK, _B, 3).transpose(0, 2, 1)
    embf = jnp.transpose(embeddings, (0, 2, 1)).reshape(-1)
    hflat = _encode(xb3.reshape(-1), embf)
    h3 = hflat.reshape(_NBLK, 4 * _NUM_LEVELS, _B)
    a0x = W0[:3].T
    a0h = W0[3:].T
    a1 = W1.T
    a2 = W2.T
    outT = _mlp(xb3, h3, a0x, a0h, a1, a2)
    return outT.T
